# baseline (speedup 1.0000x reference)
import functools

import jax
import jax.numpy as jnp
from jax import lax
from jax.experimental import pallas as pl
from jax.experimental.pallas import tpu as pltpu
from jax.experimental.pallas import tpu_sc as plsc

_N = 10000
_D = 128
_E = 320000
_NPAD = 10240
_NC = 2
_NS = 16
_NW = _NC * _NS
_CH = 128
_NCH = 80
_EPW = _CH * _NCH
_EPAD = _NW * _EPW
_STRIPE = _NPAD // _NS


def _mesh():
    return plsc.VectorSubcoreMesh(core_axis_name="c", subcore_axis_name="s")


def _unpack_chunk(pidx, ch, sidx, didx, b):
    for j in range(_D // 16):
        p = pidx[ch, pl.ds(j * 16, 16)]
        sidx[b, pl.ds(j * 16, 16)] = p & jnp.int32(0xFFFF)
        didx[b, pl.ds(j * 16, 16)] = jax.lax.shift_right_logical(
            p, jnp.int32(16))


def _deg_body(pk_hbm, out_hbm, pidx, sidx, didx, ones, zblk, deg_s, deg_d,
              sa0, sa1, sb0, sb1):
    c = lax.axis_index("c")
    s = lax.axis_index("s")
    wid = s * _NC + c
    for j in range(_CH // 16):
        ones[pl.ds(j * 16, 16)] = jnp.ones((16,), jnp.float32)
    for j in range(_STRIPE // 16):
        zblk[pl.ds(j * 16, 16)] = jnp.zeros((16,), jnp.float32)
    pltpu.sync_copy(zblk, deg_s.at[pl.ds(s * _STRIPE, _STRIPE)])
    pltpu.sync_copy(zblk, deg_d.at[pl.ds(s * _STRIPE, _STRIPE)])
    pltpu.sync_copy(pk_hbm.at[pl.ds(wid * _NCH, _NCH)], pidx)
    plsc.subcore_barrier()

    sas = (sa0, sa1)
    sbs = (sb0, sb1)

    def dstart(b):
        pltpu.async_copy(ones, deg_s.at[sidx.at[b]], sas[b], add=True)
        pltpu.async_copy(ones, deg_d.at[didx.at[b]], sbs[b], add=True)

    def dwait(b):
        pltpu.make_async_copy(ones, deg_s.at[sidx.at[b]], sas[b]).wait()
        pltpu.make_async_copy(ones, deg_d.at[didx.at[b]], sbs[b]).wait()

    for b in range(2):
        _unpack_chunk(pidx, b, sidx, didx, b)
        dstart(b)

    def body(i, carry):
        for b in range(2):
            ch = i * 2 + b
            dwait(b)
            _unpack_chunk(pidx, ch + 2, sidx, didx, b)
            dstart(b)
        return carry

    lax.fori_loop(0, _NCH // 2 - 1, body, 0)
    dwait(0)
    dwait(1)
    plsc.subcore_barrier()
    pltpu.sync_copy(deg_s.at[pl.ds(s * _STRIPE, _STRIPE)],
                    out_hbm.at[c, 0, pl.ds(s * _STRIPE, _STRIPE)])
    pltpu.sync_copy(deg_d.at[pl.ds(s * _STRIPE, _STRIPE)],
                    out_hbm.at[c, 1, pl.ds(s * _STRIPE, _STRIPE)])


def _sc_degrees(pk2):
    k = pl.kernel(
        _deg_body,
        out_type=jax.ShapeDtypeStruct((_NC, 2, _NPAD), jnp.float32),
        mesh=_mesh(),
        scratch_types=[
            pltpu.VMEM((_NCH, _CH), jnp.int32),
            pltpu.VMEM((2, _CH), jnp.int32),
            pltpu.VMEM((2, _CH), jnp.int32),
            pltpu.VMEM((_CH,), jnp.float32),
            pltpu.VMEM((_STRIPE,), jnp.float32),
            pltpu.VMEM_SHARED((_NPAD,), jnp.float32),
            pltpu.VMEM_SHARED((_NPAD,), jnp.float32),
            pltpu.SemaphoreType.DMA,
            pltpu.SemaphoreType.DMA,
            pltpu.SemaphoreType.DMA,
            pltpu.SemaphoreType.DMA,
        ],
    )
    return k(pk2)


def _agg_body(tab_hbm, pk_hbm, out_hbm, pidx, sidx, didx, rows, agg,
              sg0, sg1, ss0, ss1):
    c = lax.axis_index("c")
    s = lax.axis_index("s")
    wid = s * _NC + c

    def zrow(r, carry):
        for j in range(_D // 16):
            rows[0, r, pl.ds(j * 16, 16)] = jnp.zeros((16,), jnp.float32)
        return carry

    lax.fori_loop(0, _CH, zrow, 0)

    def zcopy(kk, carry):
        pltpu.sync_copy(rows.at[0], agg.at[pl.ds(s * _STRIPE + kk * _CH, _CH)])
        return carry

    lax.fori_loop(0, _STRIPE // _CH, zcopy, 0)
    pltpu.sync_copy(pk_hbm.at[pl.ds(wid * _NCH, _NCH)], pidx)
    plsc.subcore_barrier()

    sgs = (sg0, sg1)
    sss = (ss0, ss1)

    def gstart(b):
        pltpu.async_copy(tab_hbm.at[sidx.at[b]], rows.at[b], sgs[b])

    def gwait(b):
        pltpu.make_async_copy(tab_hbm.at[sidx.at[b]], rows.at[b],
                              sgs[b]).wait()

    def sstart(b):
        pltpu.async_copy(rows.at[b], agg.at[didx.at[b]], sss[b], add=True)

    def swait(b):
        pltpu.make_async_copy(rows.at[b], agg.at[didx.at[b]], sss[b]).wait()

    _unpack_chunk(pidx, 0, sidx, didx, 0)
    gstart(0)
    _unpack_chunk(pidx, 1, sidx, didx, 1)
    gstart(1)
    gwait(0)
    sstart(0)

    def body(k, carry):
        ch1 = 2 * k + 1
        gwait(1)
        sstart(1)
        swait(0)
        _unpack_chunk(pidx, ch1 + 1, sidx, didx, 0)
        gstart(0)
        gwait(0)
        sstart(0)
        swait(1)
        _unpack_chunk(pidx, ch1 + 2, sidx, didx, 1)
        gstart(1)
        return carry

    lax.fori_loop(0, (_NCH - 2) // 2, body, 0)
    gwait(1)
    sstart(1)
    swait(0)
    swait(1)
    plsc.subcore_barrier()
    pltpu.sync_copy(agg.at[pl.ds(s * _STRIPE, _STRIPE)],
                    out_hbm.at[c, pl.ds(s * _STRIPE, _STRIPE)])


def _sc_agg(table, pk2):
    k = pl.kernel(
        _agg_body,
        out_type=jax.ShapeDtypeStruct((_NC, _NPAD, _D), jnp.float32),
        mesh=_mesh(),
        scratch_types=[
            pltpu.VMEM((_NCH, _CH), jnp.int32),
            pltpu.VMEM((2, _CH), jnp.int32),
            pltpu.VMEM((2, _CH), jnp.int32),
            pltpu.VMEM((2, _CH, _D), jnp.float32),
            pltpu.VMEM_SHARED((_NPAD, _D), jnp.float32),
            pltpu.SemaphoreType.DMA,
            pltpu.SemaphoreType.DMA,
            pltpu.SemaphoreType.DMA,
            pltpu.SemaphoreType.DMA,
        ],
    )
    return k(table, pk2)


_TB = 1024


def _prep_tc(x, deg):
    def body(x_ref, deg_ref, xn_ref, ns_ref, nd_ref):
        dg = deg_ref[...]
        ns = lax.rsqrt(jnp.maximum(dg[0, 0] + dg[1, 0], 1.0))
        nd = lax.rsqrt(jnp.maximum(dg[0, 1] + dg[1, 1], 1.0))
        ns_ref[...] = ns
        nd_ref[...] = nd
        xn_ref[...] = x_ref[...] * ns[:, None]

    return pl.pallas_call(
        body,
        grid=(_NPAD // _TB,),
        in_specs=[
            pl.BlockSpec((_TB, _D), lambda i: (i, 0)),
            pl.BlockSpec((_NC, 2, _TB), lambda i: (0, 0, i)),
        ],
        out_specs=[
            pl.BlockSpec((_TB, _D), lambda i: (i, 0)),
            pl.BlockSpec((_TB,), lambda i: (i,)),
            pl.BlockSpec((_TB,), lambda i: (i,)),
        ],
        out_shape=[
            jax.ShapeDtypeStruct((_NPAD, _D), jnp.float32),
            jax.ShapeDtypeStruct((_NPAD,), jnp.float32),
            jax.ShapeDtypeStruct((_NPAD,), jnp.float32),
        ],
    )(x, deg)


def _layer1_tc(p1, x, ns, nd, W1, b1):
    def body(p_ref, x_ref, ns_ref, nd_ref, w_ref, b_ref, hn_ref):
        agg = p_ref[0] + p_ref[1]
        rst = agg * nd_ref[...][:, None]
        out1 = jnp.dot(rst, w_ref[...], preferred_element_type=jnp.float32)
        out1 = out1 + b_ref[...][None, :]
        h = jnp.maximum(out1, 0.0) + x_ref[...]
        hn_ref[...] = h * ns_ref[...][:, None]

    return pl.pallas_call(
        body,
        grid=(_NPAD // _TB,),
        in_specs=[
            pl.BlockSpec((_NC, _TB, _D), lambda i: (0, i, 0)),
            pl.BlockSpec((_TB, _D), lambda i: (i, 0)),
            pl.BlockSpec((_TB,), lambda i: (i,)),
            pl.BlockSpec((_TB,), lambda i: (i,)),
            pl.BlockSpec((_D, _D), lambda i: (0, 0)),
            pl.BlockSpec((_D,), lambda i: (0,)),
        ],
        out_specs=pl.BlockSpec((_TB, _D), lambda i: (i, 0)),
        out_shape=jax.ShapeDtypeStruct((_NPAD, _D), jnp.float32),
    )(p1, x, ns, nd, W1, b1)


def _layer2_tc(p2, nd, W2, b2):
    def body(p_ref, nd_ref, w_ref, b_ref, out_ref):
        agg = p_ref[0] + p_ref[1]
        rst = agg * nd_ref[...][:, None]
        out = jnp.dot(rst, w_ref[...], preferred_element_type=jnp.float32)
        out_ref[...] = out + b_ref[...][None, :]

    return pl.pallas_call(
        body,
        grid=(_NPAD // _TB,),
        in_specs=[
            pl.BlockSpec((_NC, _TB, _D), lambda i: (0, i, 0)),
            pl.BlockSpec((_TB,), lambda i: (i,)),
            pl.BlockSpec((_D, _D), lambda i: (0, 0)),
            pl.BlockSpec((_D,), lambda i: (0,)),
        ],
        out_specs=pl.BlockSpec((_TB, _D), lambda i: (i, 0)),
        out_shape=jax.ShapeDtypeStruct((_N, _D), jnp.float32),
    )(p2, nd, W2, b2)


def kernel(x, edge_index, W1, b1, W2, b2):
    npad_e = _EPAD - _E
    dummy = _N + (jnp.arange(npad_e, dtype=jnp.int32) % (_NPAD - _N))
    src = jnp.concatenate([edge_index[0], dummy])
    dst = jnp.concatenate([edge_index[1], dummy])
    packed = src | (dst << jnp.int32(16))
    pk2 = packed.reshape(_EPAD // _CH, _CH)

    deg = _sc_degrees(pk2)
    xn, ns, nd = _prep_tc(x, deg)
    p1 = _sc_agg(xn, pk2)
    hn = _layer1_tc(p1, x, ns, nd, W1, b1)
    p2 = _sc_agg(hn, pk2)
    return _layer2_tc(p2, nd, W2, b2)


# speedup vs baseline: 1.1760x; 1.1760x over previous
import functools

import jax
import jax.numpy as jnp
from jax import lax
from jax.experimental import pallas as pl
from jax.experimental.pallas import tpu as pltpu
from jax.experimental.pallas import tpu_sc as plsc

_N = 10000
_D = 128
_E = 320000
_NPAD = 10240
_NC = 2
_NS = 16
_NW = _NC * _NS
_CH = 128
_NCH = 80
_EPW = _CH * _NCH
_EPAD = _NW * _EPW
_STRIPE = _NPAD // _NS


def _mesh():
    return plsc.VectorSubcoreMesh(core_axis_name="c", subcore_axis_name="s")


def _unpack_chunk(pidx, ch, sidx, didx, b):
    for j in range(_D // 16):
        p = pidx[ch, pl.ds(j * 16, 16)]
        sidx[b, pl.ds(j * 16, 16)] = p & jnp.int32(0xFFFF)
        didx[b, pl.ds(j * 16, 16)] = jax.lax.shift_right_logical(
            p, jnp.int32(16))


def _deg_body(pk_hbm, out_hbm, pidx, sidx, didx, ones, zblk, deg_s, deg_d,
              sa0, sa1, sb0, sb1):
    c = lax.axis_index("c")
    s = lax.axis_index("s")
    wid = s * _NC + c
    for j in range(_CH // 16):
        ones[pl.ds(j * 16, 16)] = jnp.ones((16,), jnp.float32)
    for j in range(_STRIPE // 16):
        zblk[pl.ds(j * 16, 16)] = jnp.zeros((16,), jnp.float32)
    pltpu.sync_copy(zblk, deg_s.at[pl.ds(s * _STRIPE, _STRIPE)])
    pltpu.sync_copy(zblk, deg_d.at[pl.ds(s * _STRIPE, _STRIPE)])
    pltpu.sync_copy(pk_hbm.at[pl.ds(wid * _NCH, _NCH)], pidx)
    plsc.subcore_barrier()

    sas = (sa0, sa1)
    sbs = (sb0, sb1)

    def dstart(b):
        pltpu.async_copy(ones, deg_s.at[sidx.at[b]], sas[b], add=True)
        pltpu.async_copy(ones, deg_d.at[didx.at[b]], sbs[b], add=True)

    def dwait(b):
        pltpu.make_async_copy(ones, deg_s.at[sidx.at[b]], sas[b]).wait()
        pltpu.make_async_copy(ones, deg_d.at[didx.at[b]], sbs[b]).wait()

    for b in range(2):
        _unpack_chunk(pidx, b, sidx, didx, b)
        dstart(b)

    def body(i, carry):
        for b in range(2):
            ch = i * 2 + b
            dwait(b)
            _unpack_chunk(pidx, ch + 2, sidx, didx, b)
            dstart(b)
        return carry

    lax.fori_loop(0, _NCH // 2 - 1, body, 0)
    dwait(0)
    dwait(1)
    plsc.subcore_barrier()
    pltpu.sync_copy(deg_s.at[pl.ds(s * _STRIPE, _STRIPE)],
                    out_hbm.at[c, 0, pl.ds(s * _STRIPE, _STRIPE)])
    pltpu.sync_copy(deg_d.at[pl.ds(s * _STRIPE, _STRIPE)],
                    out_hbm.at[c, 1, pl.ds(s * _STRIPE, _STRIPE)])


def _sc_degrees(pk2):
    k = pl.kernel(
        _deg_body,
        out_type=jax.ShapeDtypeStruct((_NC, 2, _NPAD), jnp.float32),
        mesh=_mesh(),
        scratch_types=[
            pltpu.VMEM((_NCH, _CH), jnp.int32),
            pltpu.VMEM((2, _CH), jnp.int32),
            pltpu.VMEM((2, _CH), jnp.int32),
            pltpu.VMEM((_CH,), jnp.float32),
            pltpu.VMEM((_STRIPE,), jnp.float32),
            pltpu.VMEM_SHARED((_NPAD,), jnp.float32),
            pltpu.VMEM_SHARED((_NPAD,), jnp.float32),
            pltpu.SemaphoreType.DMA,
            pltpu.SemaphoreType.DMA,
            pltpu.SemaphoreType.DMA,
            pltpu.SemaphoreType.DMA,
        ],
    )
    return k(pk2)


def _agg_body(tab_hbm, pk_hbm, out_hbm, pidx, sidx, didx, rows, agg,
              sg0, sg1):
    c = lax.axis_index("c")
    s = lax.axis_index("s")
    wid = s * _NC + c

    def zrow(r, carry):
        for j in range(_D // 16):
            rows[0, r, pl.ds(j * 16, 16)] = jnp.zeros((16,), jnp.float32)
        return carry

    lax.fori_loop(0, _CH, zrow, 0)

    def zcopy(kk, carry):
        pltpu.sync_copy(rows.at[0], agg.at[pl.ds(s * _STRIPE + kk * _CH, _CH)])
        return carry

    lax.fori_loop(0, _STRIPE // _CH, zcopy, 0)
    pltpu.sync_copy(pk_hbm.at[pl.ds(wid * _NCH, _NCH)], pidx)
    plsc.subcore_barrier()

    sgs = (sg0, sg1)

    def gstart(b):
        pltpu.async_copy(tab_hbm.at[sidx.at[b]], rows.at[b], sgs[b])

    def gwait(b):
        pltpu.make_async_copy(tab_hbm.at[sidx.at[b]], rows.at[b],
                              sgs[b]).wait()

    for b in range(2):
        _unpack_chunk(pidx, b, sidx, didx, b)
        gstart(b)

    def body(i, carry):
        for b in range(2):
            ch = i * 2 + b
            gwait(b)
            pltpu.sync_copy(rows.at[b], agg.at[didx.at[b]], add=True)
            _unpack_chunk(pidx, ch + 2, sidx, didx, b)
            gstart(b)
        return carry

    lax.fori_loop(0, _NCH // 2 - 1, body, 0)
    for b in range(2):
        gwait(b)
        pltpu.sync_copy(rows.at[b], agg.at[didx.at[b]], add=True)
    plsc.subcore_barrier()
    pltpu.sync_copy(agg.at[pl.ds(s * _STRIPE, _STRIPE)],
                    out_hbm.at[c, pl.ds(s * _STRIPE, _STRIPE)])


def _sc_agg(table, pk2):
    k = pl.kernel(
        _agg_body,
        out_type=jax.ShapeDtypeStruct((_NC, _NPAD, _D), jnp.float32),
        mesh=_mesh(),
        scratch_types=[
            pltpu.VMEM((_NCH, _CH), jnp.int32),
            pltpu.VMEM((2, _CH), jnp.int32),
            pltpu.VMEM((2, _CH), jnp.int32),
            pltpu.VMEM((2, _CH, _D), jnp.float32),
            pltpu.VMEM_SHARED((_NPAD, _D), jnp.float32),
            pltpu.SemaphoreType.DMA,
            pltpu.SemaphoreType.DMA,
        ],
    )
    return k(table, pk2)


_TB = 1024


def _prep_tc(x, deg):
    def body(x_ref, deg_ref, xn_ref, ns_ref, nd_ref):
        dg = deg_ref[...]
        ns = lax.rsqrt(jnp.maximum(dg[0, 0] + dg[1, 0], 1.0))
        nd = lax.rsqrt(jnp.maximum(dg[0, 1] + dg[1, 1], 1.0))
        ns_ref[...] = ns
        nd_ref[...] = nd
        xn_ref[...] = x_ref[...] * ns[:, None]

    return pl.pallas_call(
        body,
        grid=(_NPAD // _TB,),
        in_specs=[
            pl.BlockSpec((_TB, _D), lambda i: (i, 0)),
            pl.BlockSpec((_NC, 2, _TB), lambda i: (0, 0, i)),
        ],
        out_specs=[
            pl.BlockSpec((_TB, _D), lambda i: (i, 0)),
            pl.BlockSpec((_TB,), lambda i: (i,)),
            pl.BlockSpec((_TB,), lambda i: (i,)),
        ],
        out_shape=[
            jax.ShapeDtypeStruct((_NPAD, _D), jnp.float32),
            jax.ShapeDtypeStruct((_NPAD,), jnp.float32),
            jax.ShapeDtypeStruct((_NPAD,), jnp.float32),
        ],
    )(x, deg)


def _layer1_tc(p1, x, ns, nd, W1, b1):
    def body(p_ref, x_ref, ns_ref, nd_ref, w_ref, b_ref, hn_ref):
        agg = p_ref[0] + p_ref[1]
        rst = agg * nd_ref[...][:, None]
        out1 = jnp.dot(rst, w_ref[...], preferred_element_type=jnp.float32)
        out1 = out1 + b_ref[...][None, :]
        h = jnp.maximum(out1, 0.0) + x_ref[...]
        hn_ref[...] = h * ns_ref[...][:, None]

    return pl.pallas_call(
        body,
        grid=(_NPAD // _TB,),
        in_specs=[
            pl.BlockSpec((_NC, _TB, _D), lambda i: (0, i, 0)),
            pl.BlockSpec((_TB, _D), lambda i: (i, 0)),
            pl.BlockSpec((_TB,), lambda i: (i,)),
            pl.BlockSpec((_TB,), lambda i: (i,)),
            pl.BlockSpec((_D, _D), lambda i: (0, 0)),
            pl.BlockSpec((_D,), lambda i: (0,)),
        ],
        out_specs=pl.BlockSpec((_TB, _D), lambda i: (i, 0)),
        out_shape=jax.ShapeDtypeStruct((_NPAD, _D), jnp.float32),
    )(p1, x, ns, nd, W1, b1)


def _layer2_tc(p2, nd, W2, b2):
    def body(p_ref, nd_ref, w_ref, b_ref, out_ref):
        agg = p_ref[0] + p_ref[1]
        rst = agg * nd_ref[...][:, None]
        out = jnp.dot(rst, w_ref[...], preferred_element_type=jnp.float32)
        out_ref[...] = out + b_ref[...][None, :]

    return pl.pallas_call(
        body,
        grid=(_NPAD // _TB,),
        in_specs=[
            pl.BlockSpec((_NC, _TB, _D), lambda i: (0, i, 0)),
            pl.BlockSpec((_TB,), lambda i: (i,)),
            pl.BlockSpec((_D, _D), lambda i: (0, 0)),
            pl.BlockSpec((_D,), lambda i: (0,)),
        ],
        out_specs=pl.BlockSpec((_TB, _D), lambda i: (i, 0)),
        out_shape=jax.ShapeDtypeStruct((_N, _D), jnp.float32),
    )(p2, nd, W2, b2)


def kernel(x, edge_index, W1, b1, W2, b2):
    npad_e = _EPAD - _E
    dummy = _N + (jnp.arange(npad_e, dtype=jnp.int32) % (_NPAD - _N))
    src = jnp.concatenate([edge_index[0], dummy])
    dst = jnp.concatenate([edge_index[1], dummy])
    packed = src | (dst << jnp.int32(16))
    pk2 = packed.reshape(_EPAD // _CH, _CH)

    deg = _sc_degrees(pk2)
    xn, ns, nd = _prep_tc(x, deg)
    p1 = _sc_agg(xn, pk2)
    hn = _layer1_tc(p1, x, ns, nd, W1, b1)
    p2 = _sc_agg(hn, pk2)
    return _layer2_tc(p2, nd, W2, b2)


# speedup vs baseline: 1.2110x; 1.0297x over previous
import functools

import jax
import jax.numpy as jnp
from jax import lax
from jax.experimental import pallas as pl
from jax.experimental.pallas import tpu as pltpu
from jax.experimental.pallas import tpu_sc as plsc

_N = 10000
_D = 128
_E = 320000
_NPAD = 10240
_NC = 2
_NS = 16
_NW = _NC * _NS
_CH = 128
_NCH = 80
_EPW = _CH * _NCH
_EPAD = _NW * _EPW
_STRIPE = _NPAD // _NS


def _mesh():
    return plsc.VectorSubcoreMesh(core_axis_name="c", subcore_axis_name="s")


def _unpack_chunk(pidx, ch, sidx, didx, b):
    for j in range(_D // 16):
        p = pidx[ch, pl.ds(j * 16, 16)]
        sidx[b, pl.ds(j * 16, 16)] = p & jnp.int32(0xFFFF)
        didx[b, pl.ds(j * 16, 16)] = jax.lax.shift_right_logical(
            p, jnp.int32(16))


_RCH = _E // _CH
_FULLW = _RCH // _NCH
_LASTN = _RCH - _FULLW * _NCH


def _deg_body(e3_hbm, out_hbm, pk_hbm, sblk, dblk, pk, ones, zblk,
              deg_s, deg_d, sa0, sa1, sb0, sb1):
    c = lax.axis_index("c")
    s = lax.axis_index("s")
    wid = s * _NC + c
    nreal = jnp.where(wid < _FULLW, _NCH, _LASTN)
    for j in range(_CH // 16):
        ones[pl.ds(j * 16, 16)] = jnp.ones((16,), jnp.float32)
    for j in range(_STRIPE // 16):
        zblk[pl.ds(j * 16, 16)] = jnp.zeros((16,), jnp.float32)
    pltpu.sync_copy(zblk, deg_s.at[pl.ds(s * _STRIPE, _STRIPE)])
    pltpu.sync_copy(zblk, deg_d.at[pl.ds(s * _STRIPE, _STRIPE)])

    @pl.when(wid < _FULLW)
    def _():
        pltpu.sync_copy(e3_hbm.at[0, pl.ds(wid * _NCH, _NCH)], sblk)
        pltpu.sync_copy(e3_hbm.at[1, pl.ds(wid * _NCH, _NCH)], dblk)

    @pl.when(wid == _FULLW)
    def _():
        pltpu.sync_copy(e3_hbm.at[0, pl.ds(_FULLW * _NCH, _LASTN)],
                        sblk.at[pl.ds(0, _LASTN)])
        pltpu.sync_copy(e3_hbm.at[1, pl.ds(_FULLW * _NCH, _LASTN)],
                        dblk.at[pl.ds(0, _LASTN)])

    plsc.subcore_barrier()

    sas = (sa0, sa1)
    sbs = (sb0, sb1)

    def dstart(ch, b):
        pltpu.async_copy(ones, deg_s.at[sblk.at[ch]], sas[b], add=True)
        pltpu.async_copy(ones, deg_d.at[dblk.at[ch]], sbs[b], add=True)

    def dwait(b):
        pltpu.make_async_copy(ones, deg_s.at[sblk.at[0]], sas[b]).wait()
        pltpu.make_async_copy(ones, deg_d.at[dblk.at[0]], sbs[b]).wait()

    def pack(ch):
        for j in range(_CH // 16):
            pk[ch, pl.ds(j * 16, 16)] = sblk[ch, pl.ds(j * 16, 16)] | (
                dblk[ch, pl.ds(j * 16, 16)] << jnp.int32(16))

    dstart(0, 0)
    dstart(1, 1)
    pack(0)
    pack(1)

    def body(i, carry):
        dwait(0)
        dstart(2 * i, 0)
        dwait(1)
        dstart(2 * i + 1, 1)
        pack(2 * i)
        pack(2 * i + 1)
        return carry

    lax.fori_loop(1, nreal // 2, body, 0)
    dwait(0)
    dwait(1)

    def dummy(i, carry):
        base = jnp.int32(_N) + (i * _CH) % jnp.int32(_NPAD - _N)
        lane = lax.iota(jnp.int32, 16)
        for j in range(_CH // 16):
            v = jnp.int32(_N) + (base - _N + j * 16 + lane) % jnp.int32(
                _NPAD - _N)
            pk[i, pl.ds(j * 16, 16)] = v | (v << jnp.int32(16))
        return carry

    lax.fori_loop(nreal, _NCH, dummy, 0)
    pltpu.sync_copy(pk, pk_hbm.at[pl.ds(wid * _NCH, _NCH)])
    plsc.subcore_barrier()
    pltpu.sync_copy(deg_s.at[pl.ds(s * _STRIPE, _STRIPE)],
                    out_hbm.at[c, 0, pl.ds(s * _STRIPE, _STRIPE)])
    pltpu.sync_copy(deg_d.at[pl.ds(s * _STRIPE, _STRIPE)],
                    out_hbm.at[c, 1, pl.ds(s * _STRIPE, _STRIPE)])


def _sc_degrees(e3):
    k = pl.kernel(
        _deg_body,
        out_type=[
            jax.ShapeDtypeStruct((_NC, 2, _NPAD), jnp.float32),
            jax.ShapeDtypeStruct((_EPAD // _CH, _CH), jnp.int32),
        ],
        mesh=_mesh(),
        scratch_types=[
            pltpu.VMEM((_NCH, _CH), jnp.int32),
            pltpu.VMEM((_NCH, _CH), jnp.int32),
            pltpu.VMEM((_NCH, _CH), jnp.int32),
            pltpu.VMEM((_CH,), jnp.float32),
            pltpu.VMEM((_STRIPE,), jnp.float32),
            pltpu.VMEM_SHARED((_NPAD,), jnp.float32),
            pltpu.VMEM_SHARED((_NPAD,), jnp.float32),
            pltpu.SemaphoreType.DMA,
            pltpu.SemaphoreType.DMA,
            pltpu.SemaphoreType.DMA,
            pltpu.SemaphoreType.DMA,
        ],
    )
    return k(e3)


def _agg_body(tab_hbm, pk_hbm, out_hbm, pidx, sidx, didx, rows, agg,
              sg0, sg1):
    c = lax.axis_index("c")
    s = lax.axis_index("s")
    wid = s * _NC + c

    def zrow(r, carry):
        for j in range(_D // 16):
            rows[0, r, pl.ds(j * 16, 16)] = jnp.zeros((16,), jnp.float32)
        return carry

    lax.fori_loop(0, _CH, zrow, 0)

    def zcopy(kk, carry):
        pltpu.sync_copy(rows.at[0], agg.at[pl.ds(s * _STRIPE + kk * _CH, _CH)])
        return carry

    lax.fori_loop(0, _STRIPE // _CH, zcopy, 0)
    pltpu.sync_copy(pk_hbm.at[pl.ds(wid * _NCH, _NCH)], pidx)
    plsc.subcore_barrier()

    sgs = (sg0, sg1)

    def gstart(b):
        pltpu.async_copy(tab_hbm.at[sidx.at[b]], rows.at[b], sgs[b])

    def gwait(b):
        pltpu.make_async_copy(tab_hbm.at[sidx.at[b]], rows.at[b],
                              sgs[b]).wait()

    for b in range(2):
        _unpack_chunk(pidx, b, sidx, didx, b)
        gstart(b)

    def body(i, carry):
        for b in range(2):
            ch = i * 2 + b
            gwait(b)
            pltpu.sync_copy(rows.at[b], agg.at[didx.at[b]], add=True)
            _unpack_chunk(pidx, ch + 2, sidx, didx, b)
            gstart(b)
        return carry

    lax.fori_loop(0, _NCH // 2 - 1, body, 0)
    for b in range(2):
        gwait(b)
        pltpu.sync_copy(rows.at[b], agg.at[didx.at[b]], add=True)
    plsc.subcore_barrier()
    pltpu.sync_copy(agg.at[pl.ds(s * _STRIPE, _STRIPE)],
                    out_hbm.at[c, pl.ds(s * _STRIPE, _STRIPE)])


def _sc_agg(table, pk2):
    k = pl.kernel(
        _agg_body,
        out_type=jax.ShapeDtypeStruct((_NC, _NPAD, _D), jnp.float32),
        mesh=_mesh(),
        scratch_types=[
            pltpu.VMEM((_NCH, _CH), jnp.int32),
            pltpu.VMEM((2, _CH), jnp.int32),
            pltpu.VMEM((2, _CH), jnp.int32),
            pltpu.VMEM((2, _CH, _D), jnp.float32),
            pltpu.VMEM_SHARED((_NPAD, _D), jnp.float32),
            pltpu.SemaphoreType.DMA,
            pltpu.SemaphoreType.DMA,
        ],
    )
    return k(table, pk2)


_TB = 1024


def _prep_tc(x, deg):
    def body(x_ref, deg_ref, xn_ref, ns_ref, nd_ref):
        dg = deg_ref[...]
        ns = lax.rsqrt(jnp.maximum(dg[0, 0] + dg[1, 0], 1.0))
        nd = lax.rsqrt(jnp.maximum(dg[0, 1] + dg[1, 1], 1.0))
        ns_ref[...] = ns
        nd_ref[...] = nd
        xn_ref[...] = x_ref[...] * ns[:, None]

    return pl.pallas_call(
        body,
        grid=(_NPAD // _TB,),
        in_specs=[
            pl.BlockSpec((_TB, _D), lambda i: (i, 0)),
            pl.BlockSpec((_NC, 2, _TB), lambda i: (0, 0, i)),
        ],
        out_specs=[
            pl.BlockSpec((_TB, _D), lambda i: (i, 0)),
            pl.BlockSpec((_TB,), lambda i: (i,)),
            pl.BlockSpec((_TB,), lambda i: (i,)),
        ],
        out_shape=[
            jax.ShapeDtypeStruct((_NPAD, _D), jnp.float32),
            jax.ShapeDtypeStruct((_NPAD,), jnp.float32),
            jax.ShapeDtypeStruct((_NPAD,), jnp.float32),
        ],
    )(x, deg)


def _layer1_tc(p1, x, ns, nd, W1, b1):
    def body(p_ref, x_ref, ns_ref, nd_ref, w_ref, b_ref, hn_ref):
        agg = p_ref[0] + p_ref[1]
        rst = agg * nd_ref[...][:, None]
        out1 = jnp.dot(rst, w_ref[...], preferred_element_type=jnp.float32)
        out1 = out1 + b_ref[...][None, :]
        h = jnp.maximum(out1, 0.0) + x_ref[...]
        hn_ref[...] = h * ns_ref[...][:, None]

    return pl.pallas_call(
        body,
        grid=(_NPAD // _TB,),
        in_specs=[
            pl.BlockSpec((_NC, _TB, _D), lambda i: (0, i, 0)),
            pl.BlockSpec((_TB, _D), lambda i: (i, 0)),
            pl.BlockSpec((_TB,), lambda i: (i,)),
            pl.BlockSpec((_TB,), lambda i: (i,)),
            pl.BlockSpec((_D, _D), lambda i: (0, 0)),
            pl.BlockSpec((_D,), lambda i: (0,)),
        ],
        out_specs=pl.BlockSpec((_TB, _D), lambda i: (i, 0)),
        out_shape=jax.ShapeDtypeStruct((_NPAD, _D), jnp.float32),
    )(p1, x, ns, nd, W1, b1)


def _layer2_tc(p2, nd, W2, b2):
    def body(p_ref, nd_ref, w_ref, b_ref, out_ref):
        agg = p_ref[0] + p_ref[1]
        rst = agg * nd_ref[...][:, None]
        out = jnp.dot(rst, w_ref[...], preferred_element_type=jnp.float32)
        out_ref[...] = out + b_ref[...][None, :]

    return pl.pallas_call(
        body,
        grid=(_NPAD // _TB,),
        in_specs=[
            pl.BlockSpec((_NC, _TB, _D), lambda i: (0, i, 0)),
            pl.BlockSpec((_TB,), lambda i: (i,)),
            pl.BlockSpec((_D, _D), lambda i: (0, 0)),
            pl.BlockSpec((_D,), lambda i: (0,)),
        ],
        out_specs=pl.BlockSpec((_TB, _D), lambda i: (i, 0)),
        out_shape=jax.ShapeDtypeStruct((_N, _D), jnp.float32),
    )(p2, nd, W2, b2)


def kernel(x, edge_index, W1, b1, W2, b2):
    e3 = edge_index.reshape(2, _E // _CH, _CH)

    deg, pk2 = _sc_degrees(e3)
    xn, ns, nd = _prep_tc(x, deg)
    p1 = _sc_agg(xn, pk2)
    hn = _layer1_tc(p1, x, ns, nd, W1, b1)
    p2 = _sc_agg(hn, pk2)
    return _layer2_tc(p2, nd, W2, b2)


# speedup vs baseline: 1.2163x; 1.0044x over previous
import functools

import jax
import jax.numpy as jnp
from jax import lax
from jax.experimental import pallas as pl
from jax.experimental.pallas import tpu as pltpu
from jax.experimental.pallas import tpu_sc as plsc

_N = 10000
_D = 128
_E = 320000
_NPAD = 10240
_NC = 2
_NS = 16
_NW = _NC * _NS
_CH = 128
_NCH = 80
_EPW = _CH * _NCH
_EPAD = _NW * _EPW
_STRIPE = _NPAD // _NS


def _mesh():
    return plsc.VectorSubcoreMesh(core_axis_name="c", subcore_axis_name="s")


def _unpack_chunk(pidx, ch, sidx, didx, b):
    for j in range(_D // 16):
        p = pidx[ch, pl.ds(j * 16, 16)]
        sidx[b, pl.ds(j * 16, 16)] = p & jnp.int32(0xFFFF)
        didx[b, pl.ds(j * 16, 16)] = jax.lax.shift_right_logical(
            p, jnp.int32(16))


_RCH = _E // _CH
_FULLW = _RCH // _NCH
_LASTN = _RCH - _FULLW * _NCH


def _deg_body(e_hbm, out_hbm, pk_hbm, sblk, dblk, srow, drow, pk, ones, zblk,
              deg_s, deg_d, sa0, sa1, sb0, sb1):
    c = lax.axis_index("c")
    s = lax.axis_index("s")
    wid = s * _NC + c
    nreal = jnp.where(wid < _FULLW, _NCH, _LASTN)
    for j in range(_CH // 16):
        ones[pl.ds(j * 16, 16)] = jnp.ones((16,), jnp.float32)
    for j in range(_STRIPE // 16):
        zblk[pl.ds(j * 16, 16)] = jnp.zeros((16,), jnp.float32)
    pltpu.sync_copy(zblk, deg_s.at[pl.ds(s * _STRIPE, _STRIPE)])
    pltpu.sync_copy(zblk, deg_d.at[pl.ds(s * _STRIPE, _STRIPE)])

    @pl.when(wid < _FULLW)
    def _():
        pltpu.sync_copy(e_hbm.at[0, pl.ds(wid * _EPW, _EPW)], sblk)
        pltpu.sync_copy(e_hbm.at[1, pl.ds(wid * _EPW, _EPW)], dblk)

    @pl.when(wid == _FULLW)
    def _():
        pltpu.sync_copy(e_hbm.at[0, pl.ds(_FULLW * _EPW, _LASTN * _CH)],
                        sblk.at[pl.ds(0, _LASTN * _CH)])
        pltpu.sync_copy(e_hbm.at[1, pl.ds(_FULLW * _EPW, _LASTN * _CH)],
                        dblk.at[pl.ds(0, _LASTN * _CH)])

    plsc.subcore_barrier()

    sas = (sa0, sa1)
    sbs = (sb0, sb1)

    def stage(ch, b):
        for j in range(_CH // 16):
            vs = sblk[pl.ds(ch * _CH + j * 16, 16)]
            vd = dblk[pl.ds(ch * _CH + j * 16, 16)]
            srow[b, pl.ds(j * 16, 16)] = vs
            drow[b, pl.ds(j * 16, 16)] = vd
            pk[ch, pl.ds(j * 16, 16)] = vs | (vd << jnp.int32(16))

    def dstart(b):
        pltpu.async_copy(ones, deg_s.at[srow.at[b]], sas[b], add=True)
        pltpu.async_copy(ones, deg_d.at[drow.at[b]], sbs[b], add=True)

    def dwait(b):
        pltpu.make_async_copy(ones, deg_s.at[srow.at[b]], sas[b]).wait()
        pltpu.make_async_copy(ones, deg_d.at[drow.at[b]], sbs[b]).wait()

    stage(0, 0)
    dstart(0)
    stage(1, 1)
    dstart(1)

    def body(i, carry):
        dwait(0)
        stage(2 * i, 0)
        dstart(0)
        dwait(1)
        stage(2 * i + 1, 1)
        dstart(1)
        return carry

    lax.fori_loop(1, nreal // 2, body, 0)
    dwait(0)
    dwait(1)

    def dummy(i, carry):
        base = jnp.int32(_N) + (i * _CH) % jnp.int32(_NPAD - _N)
        lane = lax.iota(jnp.int32, 16)
        for j in range(_CH // 16):
            v = jnp.int32(_N) + (base - _N + j * 16 + lane) % jnp.int32(
                _NPAD - _N)
            pk[i, pl.ds(j * 16, 16)] = v | (v << jnp.int32(16))
        return carry

    lax.fori_loop(nreal, _NCH, dummy, 0)
    pltpu.sync_copy(pk, pk_hbm.at[pl.ds(wid * _NCH, _NCH)])
    plsc.subcore_barrier()
    pltpu.sync_copy(deg_s.at[pl.ds(s * _STRIPE, _STRIPE)],
                    out_hbm.at[c, 0, pl.ds(s * _STRIPE, _STRIPE)])
    pltpu.sync_copy(deg_d.at[pl.ds(s * _STRIPE, _STRIPE)],
                    out_hbm.at[c, 1, pl.ds(s * _STRIPE, _STRIPE)])


def _sc_degrees(edge_index):
    k = pl.kernel(
        _deg_body,
        out_type=[
            jax.ShapeDtypeStruct((_NC, 2, _NPAD), jnp.float32),
            jax.ShapeDtypeStruct((_EPAD // _CH, _CH), jnp.int32),
        ],
        mesh=_mesh(),
        scratch_types=[
            pltpu.VMEM((_EPW,), jnp.int32),
            pltpu.VMEM((_EPW,), jnp.int32),
            pltpu.VMEM((2, _CH), jnp.int32),
            pltpu.VMEM((2, _CH), jnp.int32),
            pltpu.VMEM((_NCH, _CH), jnp.int32),
            pltpu.VMEM((_CH,), jnp.float32),
            pltpu.VMEM((_STRIPE,), jnp.float32),
            pltpu.VMEM_SHARED((_NPAD,), jnp.float32),
            pltpu.VMEM_SHARED((_NPAD,), jnp.float32),
            pltpu.SemaphoreType.DMA,
            pltpu.SemaphoreType.DMA,
            pltpu.SemaphoreType.DMA,
            pltpu.SemaphoreType.DMA,
        ],
    )
    return k(edge_index)


def _agg_body(tab_hbm, pk_hbm, out_hbm, pidx, sidx, didx, rows, agg,
              sg0, sg1):
    c = lax.axis_index("c")
    s = lax.axis_index("s")
    wid = s * _NC + c

    def zrow(r, carry):
        for j in range(_D // 16):
            rows[0, r, pl.ds(j * 16, 16)] = jnp.zeros((16,), jnp.float32)
        return carry

    lax.fori_loop(0, _CH, zrow, 0)

    def zcopy(kk, carry):
        pltpu.sync_copy(rows.at[0], agg.at[pl.ds(s * _STRIPE + kk * _CH, _CH)])
        return carry

    lax.fori_loop(0, _STRIPE // _CH, zcopy, 0)
    pltpu.sync_copy(pk_hbm.at[pl.ds(wid * _NCH, _NCH)], pidx)
    plsc.subcore_barrier()

    sgs = (sg0, sg1)

    def gstart(b):
        pltpu.async_copy(tab_hbm.at[sidx.at[b]], rows.at[b], sgs[b])

    def gwait(b):
        pltpu.make_async_copy(tab_hbm.at[sidx.at[b]], rows.at[b],
                              sgs[b]).wait()

    for b in range(2):
        _unpack_chunk(pidx, b, sidx, didx, b)
        gstart(b)

    def body(i, carry):
        for b in range(2):
            ch = i * 2 + b
            gwait(b)
            pltpu.sync_copy(rows.at[b], agg.at[didx.at[b]], add=True)
            _unpack_chunk(pidx, ch + 2, sidx, didx, b)
            gstart(b)
        return carry

    lax.fori_loop(0, _NCH // 2 - 1, body, 0)
    for b in range(2):
        gwait(b)
        pltpu.sync_copy(rows.at[b], agg.at[didx.at[b]], add=True)
    plsc.subcore_barrier()
    pltpu.sync_copy(agg.at[pl.ds(s * _STRIPE, _STRIPE)],
                    out_hbm.at[c, pl.ds(s * _STRIPE, _STRIPE)])


def _sc_agg(table, pk2):
    k = pl.kernel(
        _agg_body,
        out_type=jax.ShapeDtypeStruct((_NC, _NPAD, _D), jnp.float32),
        mesh=_mesh(),
        scratch_types=[
            pltpu.VMEM((_NCH, _CH), jnp.int32),
            pltpu.VMEM((2, _CH), jnp.int32),
            pltpu.VMEM((2, _CH), jnp.int32),
            pltpu.VMEM((2, _CH, _D), jnp.float32),
            pltpu.VMEM_SHARED((_NPAD, _D), jnp.float32),
            pltpu.SemaphoreType.DMA,
            pltpu.SemaphoreType.DMA,
        ],
    )
    return k(table, pk2)


_TB = 1024


def _prep_tc(x, deg):
    def body(x_ref, deg_ref, xn_ref, ns_ref, nd_ref):
        dg = deg_ref[...]
        ns = lax.rsqrt(jnp.maximum(dg[0, 0] + dg[1, 0], 1.0))
        nd = lax.rsqrt(jnp.maximum(dg[0, 1] + dg[1, 1], 1.0))
        ns_ref[...] = ns
        nd_ref[...] = nd
        xn_ref[...] = x_ref[...] * ns[:, None]

    return pl.pallas_call(
        body,
        grid=(_NPAD // _TB,),
        in_specs=[
            pl.BlockSpec((_TB, _D), lambda i: (i, 0)),
            pl.BlockSpec((_NC, 2, _TB), lambda i: (0, 0, i)),
        ],
        out_specs=[
            pl.BlockSpec((_TB, _D), lambda i: (i, 0)),
            pl.BlockSpec((_TB,), lambda i: (i,)),
            pl.BlockSpec((_TB,), lambda i: (i,)),
        ],
        out_shape=[
            jax.ShapeDtypeStruct((_NPAD, _D), jnp.float32),
            jax.ShapeDtypeStruct((_NPAD,), jnp.float32),
            jax.ShapeDtypeStruct((_NPAD,), jnp.float32),
        ],
    )(x, deg)


def _layer1_tc(p1, x, ns, nd, W1, b1):
    def body(p_ref, x_ref, ns_ref, nd_ref, w_ref, b_ref, hn_ref):
        agg = p_ref[0] + p_ref[1]
        rst = agg * nd_ref[...][:, None]
        out1 = jnp.dot(rst, w_ref[...], preferred_element_type=jnp.float32)
        out1 = out1 + b_ref[...][None, :]
        h = jnp.maximum(out1, 0.0) + x_ref[...]
        hn_ref[...] = h * ns_ref[...][:, None]

    return pl.pallas_call(
        body,
        grid=(_NPAD // _TB,),
        in_specs=[
            pl.BlockSpec((_NC, _TB, _D), lambda i: (0, i, 0)),
            pl.BlockSpec((_TB, _D), lambda i: (i, 0)),
            pl.BlockSpec((_TB,), lambda i: (i,)),
            pl.BlockSpec((_TB,), lambda i: (i,)),
            pl.BlockSpec((_D, _D), lambda i: (0, 0)),
            pl.BlockSpec((_D,), lambda i: (0,)),
        ],
        out_specs=pl.BlockSpec((_TB, _D), lambda i: (i, 0)),
        out_shape=jax.ShapeDtypeStruct((_NPAD, _D), jnp.float32),
    )(p1, x, ns, nd, W1, b1)


def _layer2_tc(p2, nd, W2, b2):
    def body(p_ref, nd_ref, w_ref, b_ref, out_ref):
        agg = p_ref[0] + p_ref[1]
        rst = agg * nd_ref[...][:, None]
        out = jnp.dot(rst, w_ref[...], preferred_element_type=jnp.float32)
        out_ref[...] = out + b_ref[...][None, :]

    return pl.pallas_call(
        body,
        grid=(_NPAD // _TB,),
        in_specs=[
            pl.BlockSpec((_NC, _TB, _D), lambda i: (0, i, 0)),
            pl.BlockSpec((_TB,), lambda i: (i,)),
            pl.BlockSpec((_D, _D), lambda i: (0, 0)),
            pl.BlockSpec((_D,), lambda i: (0,)),
        ],
        out_specs=pl.BlockSpec((_TB, _D), lambda i: (i, 0)),
        out_shape=jax.ShapeDtypeStruct((_N, _D), jnp.float32),
    )(p2, nd, W2, b2)


def kernel(x, edge_index, W1, b1, W2, b2):
    deg, pk2 = _sc_degrees(edge_index)
    xn, ns, nd = _prep_tc(x, deg)
    p1 = _sc_agg(xn, pk2)
    hn = _layer1_tc(p1, x, ns, nd, W1, b1)
    p2 = _sc_agg(hn, pk2)
    return _layer2_tc(p2, nd, W2, b2)


# speedup vs baseline: 1.2509x; 1.0284x over previous
import functools

import jax
import jax.numpy as jnp
from jax import lax
from jax.experimental import pallas as pl
from jax.experimental.pallas import tpu as pltpu
from jax.experimental.pallas import tpu_sc as plsc

_N = 10000
_D = 128
_E = 320000
_NPAD = 10240
_NC = 2
_NS = 16
_NW = _NC * _NS
_CH = 128
_NCH = 80
_EPW = _CH * _NCH
_EPAD = _NW * _EPW
_STRIPE = _NPAD // _NS


def _mesh():
    return plsc.VectorSubcoreMesh(core_axis_name="c", subcore_axis_name="s")


def _unpack_chunk(pidx, ch, sidx, didx, b):
    for j in range(_D // 16):
        p = pidx[ch, pl.ds(j * 16, 16)]
        sidx[b, pl.ds(j * 16, 16)] = p & jnp.int32(0xFFFF)
        didx[b, pl.ds(j * 16, 16)] = jax.lax.shift_right_logical(
            p, jnp.int32(16))


_RCH = _E // _CH
_FULLW = _RCH // _NCH
_LASTN = _RCH - _FULLW * _NCH


def _deg_body(e_hbm, out_hbm, pk_hbm, sblk, dblk, srow, drow, pk, ones, zblk,
              deg_s, deg_d, sa0, sa1, sb0, sb1):
    c = lax.axis_index("c")
    s = lax.axis_index("s")
    wid = s * _NC + c
    nreal = jnp.where(wid < _FULLW, _NCH, _LASTN)
    for j in range(_CH // 16):
        ones[pl.ds(j * 16, 16)] = jnp.ones((16,), jnp.float32)
    for j in range(_STRIPE // 16):
        zblk[pl.ds(j * 16, 16)] = jnp.zeros((16,), jnp.float32)
    pltpu.sync_copy(zblk, deg_s.at[pl.ds(s * _STRIPE, _STRIPE)])
    pltpu.sync_copy(zblk, deg_d.at[pl.ds(s * _STRIPE, _STRIPE)])

    @pl.when(wid < _FULLW)
    def _():
        pltpu.sync_copy(e_hbm.at[0, pl.ds(wid * _EPW, _EPW)], sblk)
        pltpu.sync_copy(e_hbm.at[1, pl.ds(wid * _EPW, _EPW)], dblk)

    @pl.when(wid == _FULLW)
    def _():
        pltpu.sync_copy(e_hbm.at[0, pl.ds(_FULLW * _EPW, _LASTN * _CH)],
                        sblk.at[pl.ds(0, _LASTN * _CH)])
        pltpu.sync_copy(e_hbm.at[1, pl.ds(_FULLW * _EPW, _LASTN * _CH)],
                        dblk.at[pl.ds(0, _LASTN * _CH)])

    plsc.subcore_barrier()

    sas = (sa0, sa1)
    sbs = (sb0, sb1)

    def stage(ch, b):
        for j in range(_CH // 16):
            vs = sblk[pl.ds(ch * _CH + j * 16, 16)]
            vd = dblk[pl.ds(ch * _CH + j * 16, 16)]
            srow[b, pl.ds(j * 16, 16)] = vs
            drow[b, pl.ds(j * 16, 16)] = vd
            pk[ch, pl.ds(j * 16, 16)] = vs | (vd << jnp.int32(16))

    def dstart(b):
        pltpu.async_copy(ones, deg_s.at[srow.at[b]], sas[b], add=True)
        pltpu.async_copy(ones, deg_d.at[drow.at[b]], sbs[b], add=True)

    def dwait(b):
        pltpu.make_async_copy(ones, deg_s.at[srow.at[b]], sas[b]).wait()
        pltpu.make_async_copy(ones, deg_d.at[drow.at[b]], sbs[b]).wait()

    stage(0, 0)
    dstart(0)
    stage(1, 1)
    dstart(1)

    def body(i, carry):
        dwait(0)
        stage(2 * i, 0)
        dstart(0)
        dwait(1)
        stage(2 * i + 1, 1)
        dstart(1)
        return carry

    lax.fori_loop(1, nreal // 2, body, 0)
    dwait(0)
    dwait(1)

    def dummy(i, carry):
        base = jnp.int32(_N) + (i * _CH) % jnp.int32(_NPAD - _N)
        lane = lax.iota(jnp.int32, 16)
        for j in range(_CH // 16):
            v = jnp.int32(_N) + (base - _N + j * 16 + lane) % jnp.int32(
                _NPAD - _N)
            pk[i, pl.ds(j * 16, 16)] = v | (v << jnp.int32(16))
        return carry

    lax.fori_loop(nreal, _NCH, dummy, 0)
    pltpu.sync_copy(pk, pk_hbm.at[pl.ds(wid * _NCH, _NCH)])
    plsc.subcore_barrier()
    pltpu.sync_copy(deg_s.at[pl.ds(s * _STRIPE, _STRIPE)],
                    out_hbm.at[c, 0, pl.ds(s * _STRIPE, _STRIPE)])
    pltpu.sync_copy(deg_d.at[pl.ds(s * _STRIPE, _STRIPE)],
                    out_hbm.at[c, 1, pl.ds(s * _STRIPE, _STRIPE)])


def _sc_degrees(edge_index):
    k = pl.kernel(
        _deg_body,
        out_type=[
            jax.ShapeDtypeStruct((_NC, 2, _NPAD), jnp.float32),
            jax.ShapeDtypeStruct((_EPAD // _CH, _CH), jnp.int32),
        ],
        mesh=_mesh(),
        scratch_types=[
            pltpu.VMEM((_EPW,), jnp.int32),
            pltpu.VMEM((_EPW,), jnp.int32),
            pltpu.VMEM((2, _CH), jnp.int32),
            pltpu.VMEM((2, _CH), jnp.int32),
            pltpu.VMEM((_NCH, _CH), jnp.int32),
            pltpu.VMEM((_CH,), jnp.float32),
            pltpu.VMEM((_STRIPE,), jnp.float32),
            pltpu.VMEM_SHARED((_NPAD,), jnp.float32),
            pltpu.VMEM_SHARED((_NPAD,), jnp.float32),
            pltpu.SemaphoreType.DMA,
            pltpu.SemaphoreType.DMA,
            pltpu.SemaphoreType.DMA,
            pltpu.SemaphoreType.DMA,
        ],
    )
    return k(edge_index)


def _agg_body(tab_hbm, pk_hbm, out_hbm, pidx, sidx, didx, rows, agg,
              sg0, sg1):
    c = lax.axis_index("c")
    s = lax.axis_index("s")
    wid = s * _NC + c

    def zrow(r, carry):
        for j in range(_D // 16):
            rows[0, r, pl.ds(j * 16, 16)] = jnp.zeros((16,), jnp.float32)
        return carry

    lax.fori_loop(0, _CH, zrow, 0)

    def zcopy(kk, carry):
        pltpu.sync_copy(rows.at[0], agg.at[pl.ds(s * _STRIPE + kk * _CH, _CH)])
        return carry

    lax.fori_loop(0, _STRIPE // _CH, zcopy, 0)
    pltpu.sync_copy(pk_hbm.at[pl.ds(wid * _NCH, _NCH)], pidx)
    plsc.subcore_barrier()

    sgs = (sg0, sg1)

    def gstart(b):
        pltpu.async_copy(tab_hbm.at[sidx.at[b]], rows.at[b], sgs[b])

    def gwait(b):
        pltpu.make_async_copy(tab_hbm.at[sidx.at[b]], rows.at[b],
                              sgs[b]).wait()

    for b in range(2):
        _unpack_chunk(pidx, b, sidx, didx, b)
        gstart(b)

    def body(i, carry):
        for b in range(2):
            ch = i * 2 + b
            gwait(b)
            pltpu.sync_copy(rows.at[b], agg.at[didx.at[b]], add=True)
            _unpack_chunk(pidx, ch + 2, sidx, didx, b)
            gstart(b)
        return carry

    lax.fori_loop(0, _NCH // 2 - 1, body, 0)
    for b in range(2):
        gwait(b)
        pltpu.sync_copy(rows.at[b], agg.at[didx.at[b]], add=True)
    plsc.subcore_barrier()
    pltpu.sync_copy(agg.at[pl.ds(s * _STRIPE, _STRIPE)],
                    out_hbm.at[c, pl.ds(s * _STRIPE, _STRIPE)])


def _sc_agg(table, pk2):
    k = pl.kernel(
        _agg_body,
        out_type=jax.ShapeDtypeStruct((_NC, _NPAD, _D), jnp.float32),
        mesh=_mesh(),
        scratch_types=[
            pltpu.VMEM((_NCH, _CH), jnp.int32),
            pltpu.VMEM((2, _CH), jnp.int32),
            pltpu.VMEM((2, _CH), jnp.int32),
            pltpu.VMEM((2, _CH, _D), jnp.float32),
            pltpu.VMEM_SHARED((_NPAD, _D), jnp.float32),
            pltpu.SemaphoreType.DMA,
            pltpu.SemaphoreType.DMA,
        ],
    )
    return k(table, pk2)


_TB = 2048


def _prep_tc(x, deg):
    def body(x_ref, deg_ref, xn_ref, ns_ref, nd_ref):
        dg = deg_ref[...]
        ns = lax.rsqrt(jnp.maximum(dg[0, 0] + dg[1, 0], 1.0))
        nd = lax.rsqrt(jnp.maximum(dg[0, 1] + dg[1, 1], 1.0))
        ns_ref[...] = ns
        nd_ref[...] = nd
        xn_ref[...] = x_ref[...] * ns[:, None]

    return pl.pallas_call(
        body,
        grid=(_NPAD // _TB,),
        in_specs=[
            pl.BlockSpec((_TB, _D), lambda i: (i, 0)),
            pl.BlockSpec((_NC, 2, _TB), lambda i: (0, 0, i)),
        ],
        out_specs=[
            pl.BlockSpec((_TB, _D), lambda i: (i, 0)),
            pl.BlockSpec((_TB,), lambda i: (i,)),
            pl.BlockSpec((_TB,), lambda i: (i,)),
        ],
        out_shape=[
            jax.ShapeDtypeStruct((_NPAD, _D), jnp.float32),
            jax.ShapeDtypeStruct((_NPAD,), jnp.float32),
            jax.ShapeDtypeStruct((_NPAD,), jnp.float32),
        ],
    )(x, deg)


def _layer1_tc(p1, x, ns, nd, W1, b1):
    def body(p_ref, x_ref, ns_ref, nd_ref, w_ref, b_ref, hn_ref):
        agg = p_ref[0] + p_ref[1]
        rst = agg * nd_ref[...][:, None]
        out1 = jnp.dot(rst, w_ref[...], preferred_element_type=jnp.float32)
        out1 = out1 + b_ref[...][None, :]
        h = jnp.maximum(out1, 0.0) + x_ref[...]
        hn_ref[...] = h * ns_ref[...][:, None]

    return pl.pallas_call(
        body,
        grid=(_NPAD // _TB,),
        in_specs=[
            pl.BlockSpec((_NC, _TB, _D), lambda i: (0, i, 0)),
            pl.BlockSpec((_TB, _D), lambda i: (i, 0)),
            pl.BlockSpec((_TB,), lambda i: (i,)),
            pl.BlockSpec((_TB,), lambda i: (i,)),
            pl.BlockSpec((_D, _D), lambda i: (0, 0)),
            pl.BlockSpec((_D,), lambda i: (0,)),
        ],
        out_specs=pl.BlockSpec((_TB, _D), lambda i: (i, 0)),
        out_shape=jax.ShapeDtypeStruct((_NPAD, _D), jnp.float32),
    )(p1, x, ns, nd, W1, b1)


def _layer2_tc(p2, nd, W2, b2):
    def body(p_ref, nd_ref, w_ref, b_ref, out_ref):
        agg = p_ref[0] + p_ref[1]
        rst = agg * nd_ref[...][:, None]
        out = jnp.dot(rst, w_ref[...], preferred_element_type=jnp.float32)
        out_ref[...] = out + b_ref[...][None, :]

    return pl.pallas_call(
        body,
        grid=(_NPAD // _TB,),
        in_specs=[
            pl.BlockSpec((_NC, _TB, _D), lambda i: (0, i, 0)),
            pl.BlockSpec((_TB,), lambda i: (i,)),
            pl.BlockSpec((_D, _D), lambda i: (0, 0)),
            pl.BlockSpec((_D,), lambda i: (0,)),
        ],
        out_specs=pl.BlockSpec((_TB, _D), lambda i: (i, 0)),
        out_shape=jax.ShapeDtypeStruct((_N, _D), jnp.float32),
    )(p2, nd, W2, b2)


def kernel(x, edge_index, W1, b1, W2, b2):
    deg, pk2 = _sc_degrees(edge_index)
    xn, ns, nd = _prep_tc(x, deg)
    p1 = _sc_agg(xn, pk2)
    hn = _layer1_tc(p1, x, ns, nd, W1, b1)
    p2 = _sc_agg(hn, pk2)
    return _layer2_tc(p2, nd, W2, b2)


# speedup vs baseline: 1.2510x; 1.0001x over previous
import jax
import jax.numpy as jnp
from jax import lax
from jax.experimental import pallas as pl
from jax.experimental.pallas import tpu as pltpu
from jax.experimental.pallas import tpu_sc as plsc

_N = 10000
_D = 128
_E = 320000
_NPAD = 10240
_NC = 2
_NS = 16
_NW = _NC * _NS
_CH = 128
_NCH = 80
_EPW = _CH * _NCH
_EPAD = _NW * _EPW
_STRIPE = _NPAD // _NS


def _mesh():
    return plsc.VectorSubcoreMesh(core_axis_name="c", subcore_axis_name="s")


def _unpack_chunk(pidx, ch, sidx, didx, b):
    for j in range(_D // 16):
        p = pidx[ch, pl.ds(j * 16, 16)]
        sidx[b, pl.ds(j * 16, 16)] = p & jnp.int32(0xFFFF)
        didx[b, pl.ds(j * 16, 16)] = jax.lax.shift_right_logical(
            p, jnp.int32(16))


_RCH = _E // _CH
_FULLW = _RCH // _NCH
_LASTN = _RCH - _FULLW * _NCH


def _deg_body(e_hbm, out_hbm, pk_hbm, sblk, dblk, srow, drow, pk, ones, zblk,
              deg_s, deg_d, sa0, sa1, sb0, sb1):
    c = lax.axis_index("c")
    s = lax.axis_index("s")
    wid = s * _NC + c
    nreal = jnp.where(wid < _FULLW, _NCH, _LASTN)
    for j in range(_CH // 16):
        ones[pl.ds(j * 16, 16)] = jnp.ones((16,), jnp.float32)
    for j in range(_STRIPE // 16):
        zblk[pl.ds(j * 16, 16)] = jnp.zeros((16,), jnp.float32)
    pltpu.sync_copy(zblk, deg_s.at[pl.ds(s * _STRIPE, _STRIPE)])
    pltpu.sync_copy(zblk, deg_d.at[pl.ds(s * _STRIPE, _STRIPE)])

    @pl.when(wid < _FULLW)
    def _():
        pltpu.sync_copy(e_hbm.at[0, pl.ds(wid * _EPW, _EPW)], sblk)
        pltpu.sync_copy(e_hbm.at[1, pl.ds(wid * _EPW, _EPW)], dblk)

    @pl.when(wid == _FULLW)
    def _():
        pltpu.sync_copy(e_hbm.at[0, pl.ds(_FULLW * _EPW, _LASTN * _CH)],
                        sblk.at[pl.ds(0, _LASTN * _CH)])
        pltpu.sync_copy(e_hbm.at[1, pl.ds(_FULLW * _EPW, _LASTN * _CH)],
                        dblk.at[pl.ds(0, _LASTN * _CH)])

    plsc.subcore_barrier()

    sas = (sa0, sa1)
    sbs = (sb0, sb1)

    def stage(ch, b):
        for j in range(_CH // 16):
            vs = sblk[pl.ds(ch * _CH + j * 16, 16)]
            vd = dblk[pl.ds(ch * _CH + j * 16, 16)]
            srow[b, pl.ds(j * 16, 16)] = vs
            drow[b, pl.ds(j * 16, 16)] = vd
            pk[ch, pl.ds(j * 16, 16)] = vs | (vd << jnp.int32(16))

    def dstart(b):
        pltpu.async_copy(ones, deg_s.at[srow.at[b]], sas[b], add=True)
        pltpu.async_copy(ones, deg_d.at[drow.at[b]], sbs[b], add=True)

    def dwait(b):
        pltpu.make_async_copy(ones, deg_s.at[srow.at[b]], sas[b]).wait()
        pltpu.make_async_copy(ones, deg_d.at[drow.at[b]], sbs[b]).wait()

    stage(0, 0)
    dstart(0)
    stage(1, 1)
    dstart(1)

    def body(i, carry):
        dwait(0)
        stage(2 * i, 0)
        dstart(0)
        dwait(1)
        stage(2 * i + 1, 1)
        dstart(1)
        return carry

    lax.fori_loop(1, nreal // 2, body, 0)
    dwait(0)
    dwait(1)

    def dummy(i, carry):
        base = jnp.int32(_N) + (i * _CH) % jnp.int32(_NPAD - _N)
        lane = lax.iota(jnp.int32, 16)
        for j in range(_CH // 16):
            v = jnp.int32(_N) + (base - _N + j * 16 + lane) % jnp.int32(
                _NPAD - _N)
            pk[i, pl.ds(j * 16, 16)] = v | (v << jnp.int32(16))
        return carry

    lax.fori_loop(nreal, _NCH, dummy, 0)
    pltpu.sync_copy(pk, pk_hbm.at[pl.ds(wid * _NCH, _NCH)])
    plsc.subcore_barrier()
    pltpu.sync_copy(deg_s.at[pl.ds(s * _STRIPE, _STRIPE)],
                    out_hbm.at[c, 0, pl.ds(s * _STRIPE, _STRIPE)])
    pltpu.sync_copy(deg_d.at[pl.ds(s * _STRIPE, _STRIPE)],
                    out_hbm.at[c, 1, pl.ds(s * _STRIPE, _STRIPE)])


def _sc_degrees(edge_index):
    k = pl.kernel(
        _deg_body,
        out_type=[
            jax.ShapeDtypeStruct((_NC, 2, _NPAD), jnp.float32),
            jax.ShapeDtypeStruct((_EPAD // _CH, _CH), jnp.int32),
        ],
        mesh=_mesh(),
        scratch_types=[
            pltpu.VMEM((_EPW,), jnp.int32),
            pltpu.VMEM((_EPW,), jnp.int32),
            pltpu.VMEM((2, _CH), jnp.int32),
            pltpu.VMEM((2, _CH), jnp.int32),
            pltpu.VMEM((_NCH, _CH), jnp.int32),
            pltpu.VMEM((_CH,), jnp.float32),
            pltpu.VMEM((_STRIPE,), jnp.float32),
            pltpu.VMEM_SHARED((_NPAD,), jnp.float32),
            pltpu.VMEM_SHARED((_NPAD,), jnp.float32),
            pltpu.SemaphoreType.DMA,
            pltpu.SemaphoreType.DMA,
            pltpu.SemaphoreType.DMA,
            pltpu.SemaphoreType.DMA,
        ],
    )
    return k(edge_index)


def _agg_body(tab_hbm, pk_hbm, out_hbm, pidx, sidx, didx, rows, agg,
              sg0, sg1):
    c = lax.axis_index("c")
    s = lax.axis_index("s")
    wid = s * _NC + c

    def zrow(r, carry):
        for j in range(_D // 16):
            rows[0, r, pl.ds(j * 16, 16)] = jnp.zeros((16,), jnp.float32)
        return carry

    lax.fori_loop(0, _CH, zrow, 0)

    def zcopy(kk, carry):
        pltpu.sync_copy(rows.at[0], agg.at[pl.ds(s * _STRIPE + kk * _CH, _CH)])
        return carry

    lax.fori_loop(0, _STRIPE // _CH, zcopy, 0)
    pltpu.sync_copy(pk_hbm.at[pl.ds(wid * _NCH, _NCH)], pidx)
    plsc.subcore_barrier()

    sgs = (sg0, sg1)

    def gstart(b):
        pltpu.async_copy(tab_hbm.at[sidx.at[b]], rows.at[b], sgs[b])

    def gwait(b):
        pltpu.make_async_copy(tab_hbm.at[sidx.at[b]], rows.at[b],
                              sgs[b]).wait()

    for b in range(2):
        _unpack_chunk(pidx, b, sidx, didx, b)
        gstart(b)

    def body(i, carry):
        for b in range(2):
            ch = i * 2 + b
            gwait(b)
            pltpu.sync_copy(rows.at[b], agg.at[didx.at[b]], add=True)
            _unpack_chunk(pidx, ch + 2, sidx, didx, b)
            gstart(b)
        return carry

    lax.fori_loop(0, _NCH // 2 - 1, body, 0)
    for b in range(2):
        gwait(b)
        pltpu.sync_copy(rows.at[b], agg.at[didx.at[b]], add=True)
    plsc.subcore_barrier()
    pltpu.sync_copy(agg.at[pl.ds(s * _STRIPE, _STRIPE)],
                    out_hbm.at[c, pl.ds(s * _STRIPE, _STRIPE)])


def _sc_agg(table, pk2):
    k = pl.kernel(
        _agg_body,
        out_type=jax.ShapeDtypeStruct((_NC, _NPAD, _D), jnp.float32),
        mesh=_mesh(),
        scratch_types=[
            pltpu.VMEM((_NCH, _CH), jnp.int32),
            pltpu.VMEM((2, _CH), jnp.int32),
            pltpu.VMEM((2, _CH), jnp.int32),
            pltpu.VMEM((2, _CH, _D), jnp.float32),
            pltpu.VMEM_SHARED((_NPAD, _D), jnp.float32),
            pltpu.SemaphoreType.DMA,
            pltpu.SemaphoreType.DMA,
        ],
    )
    return k(table, pk2)


_TB = 2048


def _prep_tc(x, deg):
    def body(x_ref, deg_ref, xn_ref, ns_ref, nd_ref):
        dg = deg_ref[...]
        ns = lax.rsqrt(jnp.maximum(dg[0, 0] + dg[1, 0], 1.0))
        nd = lax.rsqrt(jnp.maximum(dg[0, 1] + dg[1, 1], 1.0))
        ns_ref[...] = ns
        nd_ref[...] = nd
        xn_ref[...] = x_ref[...] * ns[:, None]

    return pl.pallas_call(
        body,
        grid=(_NPAD // _TB,),
        in_specs=[
            pl.BlockSpec((_TB, _D), lambda i: (i, 0)),
            pl.BlockSpec((_NC, 2, _TB), lambda i: (0, 0, i)),
        ],
        out_specs=[
            pl.BlockSpec((_TB, _D), lambda i: (i, 0)),
            pl.BlockSpec((_TB,), lambda i: (i,)),
            pl.BlockSpec((_TB,), lambda i: (i,)),
        ],
        out_shape=[
            jax.ShapeDtypeStruct((_NPAD, _D), jnp.float32),
            jax.ShapeDtypeStruct((_NPAD,), jnp.float32),
            jax.ShapeDtypeStruct((_NPAD,), jnp.float32),
        ],
    )(x, deg)


def _layer1_tc(p1, x, ns, nd, W1, b1):
    def body(p_ref, x_ref, ns_ref, nd_ref, w_ref, b_ref, hn_ref):
        agg = p_ref[0] + p_ref[1]
        rst = agg * nd_ref[...][:, None]
        out1 = jnp.dot(rst, w_ref[...], preferred_element_type=jnp.float32)
        out1 = out1 + b_ref[...][None, :]
        h = jnp.maximum(out1, 0.0) + x_ref[...]
        hn_ref[...] = h * ns_ref[...][:, None]

    return pl.pallas_call(
        body,
        grid=(_NPAD // _TB,),
        in_specs=[
            pl.BlockSpec((_NC, _TB, _D), lambda i: (0, i, 0)),
            pl.BlockSpec((_TB, _D), lambda i: (i, 0)),
            pl.BlockSpec((_TB,), lambda i: (i,)),
            pl.BlockSpec((_TB,), lambda i: (i,)),
            pl.BlockSpec((_D, _D), lambda i: (0, 0)),
            pl.BlockSpec((_D,), lambda i: (0,)),
        ],
        out_specs=pl.BlockSpec((_TB, _D), lambda i: (i, 0)),
        out_shape=jax.ShapeDtypeStruct((_NPAD, _D), jnp.float32),
    )(p1, x, ns, nd, W1, b1)


def _layer2_tc(p2, nd, W2, b2):
    def body(p_ref, nd_ref, w_ref, b_ref, out_ref):
        agg = p_ref[0] + p_ref[1]
        rst = agg * nd_ref[...][:, None]
        out = jnp.dot(rst, w_ref[...], preferred_element_type=jnp.float32)
        out_ref[...] = out + b_ref[...][None, :]

    return pl.pallas_call(
        body,
        grid=(_NPAD // _TB,),
        in_specs=[
            pl.BlockSpec((_NC, _TB, _D), lambda i: (0, i, 0)),
            pl.BlockSpec((_TB,), lambda i: (i,)),
            pl.BlockSpec((_D, _D), lambda i: (0, 0)),
            pl.BlockSpec((_D,), lambda i: (0,)),
        ],
        out_specs=pl.BlockSpec((_TB, _D), lambda i: (i, 0)),
        out_shape=jax.ShapeDtypeStruct((_N, _D), jnp.float32),
    )(p2, nd, W2, b2)


def kernel(x, edge_index, W1, b1, W2, b2):
    deg, pk2 = _sc_degrees(edge_index)
    xn, ns, nd = _prep_tc(x, deg)
    p1 = _sc_agg(xn, pk2)
    hn = _layer1_tc(p1, x, ns, nd, W1, b1)
    p2 = _sc_agg(hn, pk2)
    return _layer2_tc(p2, nd, W2, b2)
